# baseline (speedup 1.0000x reference)
import functools

import jax
import jax.numpy as jnp
from jax import lax
from jax.experimental import pallas as pl
from jax.experimental.pallas import tpu as pltpu
from jax.experimental.pallas import tpu_sc as plsc

_N_ATOMS = 100000
_N_SEL = 50000
_D = 128
_DC = 3
_DCP = 16
_CHUNK = 80
_NCHUNK = _N_SEL // _CHUNK
_NW = 32
_ITERS = (_NCHUNK + _NW - 1) // _NW

_mesh = plsc.VectorSubcoreMesh(core_axis_name="c", subcore_axis_name="s")


@functools.partial(
    pl.kernel,
    mesh=_mesh,
    out_type=(
        jax.ShapeDtypeStruct((_N_SEL, _D), jnp.float32),
        jax.ShapeDtypeStruct((_N_SEL, _DCP), jnp.float32),
    ),
    scratch_types=[
        pltpu.VMEM((_CHUNK,), jnp.int32),
        pltpu.VMEM((_CHUNK, _D), jnp.float32),
        pltpu.VMEM((_CHUNK, _DCP), jnp.float32),
        pltpu.SemaphoreType.DMA,
        pltpu.SemaphoreType.DMA,
    ],
    compiler_params=pltpu.CompilerParams(use_tc_tiling_on_sc=False),
)
def _gather_both(idx_hbm, atm_hbm, coord_hbm, atm_out, coord_out,
                 idx_v, rows_v, crows_v, sem_a, sem_c):
    w = lax.axis_index("s") * 2 + lax.axis_index("c")

    def body(i, carry):
        c = w + i * _NW

        @pl.when(c < _NCHUNK)
        def _():
            base = c * _CHUNK
            pltpu.sync_copy(idx_hbm.at[pl.ds(base, _CHUNK)], idx_v)
            cp_a = pltpu.async_copy(atm_hbm.at[idx_v], rows_v, sem_a)
            cp_c = pltpu.async_copy(coord_hbm.at[idx_v], crows_v, sem_c)
            cp_a.wait()
            pltpu.sync_copy(rows_v, atm_out.at[pl.ds(base, _CHUNK)])
            cp_c.wait()
            pltpu.sync_copy(crows_v, coord_out.at[pl.ds(base, _CHUNK)])

        return carry

    lax.fori_loop(0, _ITERS, body, 0)


def kernel(ent, atm, coord, mol_feat):
    e = jnp.reshape(ent, (_N_SEL,)).astype(jnp.int32)
    atm2 = jnp.reshape(atm, (_N_ATOMS, _D))
    coord2 = jnp.pad(jnp.reshape(coord, (_N_ATOMS, _DC)),
                     ((0, 0), (0, _DCP - _DC)))
    atm_sel, coord_sel = _gather_both(e, atm2, coord2)
    return (atm_sel[None], coord_sel[None, :, :_DC], mol_feat)


# speedup vs baseline: 1.1434x; 1.1434x over previous
import functools

import jax
import jax.numpy as jnp
from jax import lax
from jax.experimental import pallas as pl
from jax.experimental.pallas import tpu as pltpu
from jax.experimental.pallas import tpu_sc as plsc

_N_ATOMS = 100000
_N_SEL = 50000
_D = 128
_DC = 3
_DCP = 16
_CHUNK = 80
_NCHUNK = _N_SEL // _CHUNK
_NW = 32
_MAXC = 20
_BIG = 17
_NB = 8
_LAG = 4

_mesh = plsc.VectorSubcoreMesh(core_axis_name="c", subcore_axis_name="s")


def _worker_span():
    w = lax.axis_index("s") * 2 + lax.axis_index("c")
    cnt = jnp.where(w < _BIG, _MAXC, _MAXC - 1)
    start_chunk = w * (_MAXC - 1) + jnp.minimum(w, _BIG)
    return start_chunk * _CHUNK, cnt


def _load_indices(idx_hbm, idx_v, rbase, cnt):
    low = (_MAXC - 1) * _CHUNK
    pltpu.sync_copy(idx_hbm.at[pl.ds(rbase, low)], idx_v.at[pl.ds(0, low)])

    @pl.when(cnt == _MAXC)
    def _():
        pltpu.sync_copy(idx_hbm.at[pl.ds(rbase + low, _CHUNK)],
                        idx_v.at[pl.ds(low, _CHUNK)])


def _gather_pipeline(tab_hbm, out_hbm, idx_v, bufs, gsems, wsems, rbase, cnt):
    def wait_gather(b):
        pltpu.make_async_copy(tab_hbm.at[pl.ds(0, _CHUNK)], bufs.at[b],
                              gsems[b]).wait()

    def wait_wb(b):
        pltpu.make_async_copy(bufs.at[b], out_hbm.at[pl.ds(0, _CHUNK)],
                              wsems[b]).wait()

    for j in range(_MAXC + _LAG):
        jc = j - _LAG
        if 0 <= jc < _MAXC:
            b = jc % _NB

            @pl.when(jc < cnt)
            def _(jc=jc, b=b):
                wait_gather(b)
                pltpu.async_copy(bufs.at[b],
                                 out_hbm.at[pl.ds(rbase + jc * _CHUNK,
                                                  _CHUNK)],
                                 wsems[b])

        if j < _MAXC:
            b = j % _NB

            @pl.when(j < cnt)
            def _(j=j, b=b):
                if j >= _NB:
                    wait_wb(b)
                pltpu.async_copy(
                    tab_hbm.at[idx_v.at[pl.ds(j * _CHUNK, _CHUNK)]],
                    bufs.at[b], gsems[b])

    for b in range(_NB):
        wait_wb(b)


def _make_gather(dcols, untiled):
    @functools.partial(
        pl.kernel,
        mesh=_mesh,
        out_type=jax.ShapeDtypeStruct((_N_SEL, dcols), jnp.float32),
        scratch_types=(
            [pltpu.VMEM((_MAXC * _CHUNK,), jnp.int32),
             pltpu.VMEM((_NB, _CHUNK, dcols), jnp.float32)]
            + [pltpu.SemaphoreType.DMA] * (2 * _NB)
        ),
        compiler_params=(pltpu.CompilerParams(use_tc_tiling_on_sc=False)
                         if untiled else None),
    )
    def _gather(idx_hbm, tab_hbm, out_hbm, idx_v, bufs, *sems):
        gsems, wsems = sems[:_NB], sems[_NB:]
        rbase, cnt = _worker_span()
        _load_indices(idx_hbm, idx_v, rbase, cnt)
        _gather_pipeline(tab_hbm, out_hbm, idx_v, bufs, gsems, wsems,
                         rbase, cnt)

    return _gather


_gather_atm = _make_gather(_D, untiled=False)
_gather_coord = _make_gather(_DCP, untiled=True)


def kernel(ent, atm, coord, mol_feat):
    e = jnp.reshape(ent, (_N_SEL,)).astype(jnp.int32)
    atm2 = jnp.reshape(atm, (_N_ATOMS, _D))
    coord2 = jnp.pad(jnp.reshape(coord, (_N_ATOMS, _DC)),
                     ((0, 0), (0, _DCP - _DC)))
    atm_sel = _gather_atm(e, atm2)
    coord_sel = _gather_coord(e, coord2)
    return (atm_sel[None], coord_sel[None, :, :_DC], mol_feat)
